# Initial kernel scaffold; baseline (speedup 1.0000x reference)
import jax
import jax.numpy as jnp
from jax import lax
from jax.experimental import pallas as pl
from jax.experimental.pallas import tpu as pltpu
from jax.experimental.pallas import tpu_sc as plsc

USER = 6000
ITEM = 4000
N = USER + ITEM
E = 320000
D = 128
LEAKY = 0.5

NC = 2
NS = 16
NW = NC * NS
CH = 128
EPT = 10240
NCH = EPT // CH
EP = NW * EPT

RPT = N // NS
ZR = 125
NZ = RPT // ZR


def _sc_layer_body(x_hbm, src_hbm, dst_hbm, vals_hbm, part_hbm,
                   idx_s, idx_d, valc, rows, stage, accum, sem):
    cid = lax.axis_index("c")
    sid = lax.axis_index("s")
    wid = sid * NC + cid
    rbase = sid * RPT

    def zrow(i, _):
        for k in range(8):
            stage[i, pl.ds(k * 16, 16)] = jnp.zeros((16,), jnp.float32)
        return 0
    lax.fori_loop(0, ZR, zrow, 0)
    for t in range(NZ):
        pltpu.sync_copy(stage, accum.at[pl.ds(rbase + t * ZR, ZR)])
    plsc.subcore_barrier()

    def chunk(j, _):
        pltpu.sync_copy(src_hbm.at[wid, j], idx_s)
        pltpu.sync_copy(dst_hbm.at[wid, j], idx_d)
        pltpu.sync_copy(vals_hbm.at[wid, j], valc)
        pltpu.async_copy(x_hbm.at[idx_s], rows, sem).wait()

        def row(r, _):
            v = plsc.load_gather(valc, [jnp.full((16,), r, jnp.int32)])
            for k in range(8):
                sl = pl.ds(k * 16, 16)
                rows[r, sl] = rows[r, sl] * v
            return 0
        lax.fori_loop(0, CH, row, 0)
        pltpu.sync_copy(rows, accum.at[idx_d], add=True)
        return 0
    lax.fori_loop(0, NCH, chunk, 0)
    plsc.subcore_barrier()

    for t in range(NZ):
        pltpu.sync_copy(accum.at[pl.ds(rbase + t * ZR, ZR)], stage)
        pltpu.sync_copy(stage, part_hbm.at[cid, pl.ds(rbase + t * ZR, ZR)])


_sc_layer = pl.kernel(
    _sc_layer_body,
    out_type=jax.ShapeDtypeStruct((NC, N, D), jnp.float32),
    mesh=plsc.VectorSubcoreMesh(core_axis_name="c", subcore_axis_name="s",
                                num_cores=NC, num_subcores=NS),
    scratch_types=[
        pltpu.VMEM((CH,), jnp.int32),
        pltpu.VMEM((CH,), jnp.int32),
        pltpu.VMEM((CH,), jnp.float32),
        pltpu.VMEM((CH, D), jnp.float32),
        pltpu.VMEM((ZR, D), jnp.float32),
        pltpu.VMEM_SHARED((N, D), jnp.float32),
        pltpu.SemaphoreType.DMA,
    ],
)


def _combine_body(p_ref, t_ref, y_ref, tot_ref):
    s = p_ref[0] + p_ref[1]
    y = jnp.maximum(s, LEAKY * s)
    y_ref[...] = y
    tot_ref[...] = t_ref[...] + y


_BR = 1000


def _combine(partial, total):
    return pl.pallas_call(
        _combine_body,
        grid=(N // _BR,),
        in_specs=[
            pl.BlockSpec((NC, _BR, D), lambda i: (0, i, 0)),
            pl.BlockSpec((_BR, D), lambda i: (i, 0)),
        ],
        out_specs=[
            pl.BlockSpec((_BR, D), lambda i: (i, 0)),
            pl.BlockSpec((_BR, D), lambda i: (i, 0)),
        ],
        out_shape=[
            jax.ShapeDtypeStruct((N, D), jnp.float32),
            jax.ShapeDtypeStruct((N, D), jnp.float32),
        ],
    )(partial, total)


def kernel(uEmbeds, iEmbeds, edge_vals, edge_index):
    x0 = jnp.concatenate([uEmbeds, iEmbeds], axis=0)
    dst = edge_index[0].astype(jnp.int32)
    src = edge_index[1].astype(jnp.int32)
    vals = edge_vals.astype(jnp.float32)
    pad = EP - E
    src = jnp.pad(src, (0, pad)).reshape(NW, NCH, CH)
    dst = jnp.pad(dst, (0, pad)).reshape(NW, NCH, CH)
    vals = jnp.pad(vals, (0, pad)).reshape(NW, NCH, CH)

    x = x0
    total = x0
    for _ in range(2):
        partial = _sc_layer(x, src, dst, vals)
        x, total = _combine(partial, total)
    return total[:USER], total[USER:]


# speedup vs baseline: 2.7257x; 2.7257x over previous
import jax
import jax.numpy as jnp
from jax import lax
from jax.experimental import pallas as pl
from jax.experimental.pallas import tpu as pltpu
from jax.experimental.pallas import tpu_sc as plsc

USER = 6000
ITEM = 4000
N = USER + ITEM
E = 320000
D = 128
LEAKY = 0.5

NC = 2
NS = 16
NW = NC * NS
CH = 128
EPT = 10240
NCH = EPT // CH
EP = NW * EPT

NP = 10240
RPT = NP // NS
ZR = 128
NZ = RPT // ZR


def _sc_layer_body(x_hbm, src_hbm, dst_hbm, vals_hbm, part_hbm,
                   idx_s, idx_d, valc, rows, stage, accum, sem):
    cid = lax.axis_index("c")
    sid = lax.axis_index("s")
    wid = sid * NC + cid
    rbase = sid * RPT

    def zrow(i, _):
        for k in range(8):
            stage[i, pl.ds(k * 16, 16)] = jnp.zeros((16,), jnp.float32)
        return 0
    lax.fori_loop(0, ZR, zrow, 0)
    for t in range(NZ):
        pltpu.sync_copy(stage, accum.at[pl.ds(rbase + t * ZR, ZR)])
    plsc.subcore_barrier()

    def chunk(j, _):
        pltpu.sync_copy(src_hbm.at[wid, j], idx_s)
        pltpu.sync_copy(dst_hbm.at[wid, j], idx_d)
        pltpu.sync_copy(vals_hbm.at[wid, j], valc)
        pltpu.async_copy(x_hbm.at[idx_s], rows, sem).wait()

        def row16(g, _):
            base = g * 16
            vals16 = valc[pl.ds(base, 16)]
            for r in range(16):
                v = jnp.full((16,), vals16[r], jnp.float32)
                for k in range(8):
                    sl = pl.ds(k * 16, 16)
                    rows[base + r, sl] = rows[base + r, sl] * v
            return 0
        lax.fori_loop(0, CH // 16, row16, 0)
        pltpu.sync_copy(rows, accum.at[idx_d], add=True)
        return 0
    lax.fori_loop(0, NCH, chunk, 0)
    plsc.subcore_barrier()

    for t in range(NZ):
        pltpu.sync_copy(accum.at[pl.ds(rbase + t * ZR, ZR)], stage)
        pltpu.sync_copy(stage, part_hbm.at[cid, pl.ds(rbase + t * ZR, ZR)])


_sc_layer = pl.kernel(
    _sc_layer_body,
    out_type=jax.ShapeDtypeStruct((NC, NP, D), jnp.float32),
    mesh=plsc.VectorSubcoreMesh(core_axis_name="c", subcore_axis_name="s",
                                num_cores=NC, num_subcores=NS),
    scratch_types=[
        pltpu.VMEM((CH,), jnp.int32),
        pltpu.VMEM((CH,), jnp.int32),
        pltpu.VMEM((CH,), jnp.float32),
        pltpu.VMEM((CH, D), jnp.float32),
        pltpu.VMEM((ZR, D), jnp.float32),
        pltpu.VMEM_SHARED((NP, D), jnp.float32),
        pltpu.SemaphoreType.DMA,
    ],
)


def _combine_body(p_ref, t_ref, y_ref, tot_ref):
    s = p_ref[0] + p_ref[1]
    y = jnp.maximum(s, LEAKY * s)
    y_ref[...] = y
    tot_ref[...] = t_ref[...] + y


_BR = 1024


def _combine(partial, total):
    return pl.pallas_call(
        _combine_body,
        grid=(NP // _BR,),
        in_specs=[
            pl.BlockSpec((NC, _BR, D), lambda i: (0, i, 0)),
            pl.BlockSpec((_BR, D), lambda i: (i, 0)),
        ],
        out_specs=[
            pl.BlockSpec((_BR, D), lambda i: (i, 0)),
            pl.BlockSpec((_BR, D), lambda i: (i, 0)),
        ],
        out_shape=[
            jax.ShapeDtypeStruct((NP, D), jnp.float32),
            jax.ShapeDtypeStruct((NP, D), jnp.float32),
        ],
    )(partial, total)


def kernel(uEmbeds, iEmbeds, edge_vals, edge_index):
    x0 = jnp.concatenate([uEmbeds, iEmbeds], axis=0)
    dst = edge_index[0].astype(jnp.int32)
    src = edge_index[1].astype(jnp.int32)
    vals = edge_vals.astype(jnp.float32)
    pad = EP - E
    src = jnp.pad(src, (0, pad)).reshape(NW, NCH, CH)
    dst = jnp.pad(dst, (0, pad)).reshape(NW, NCH, CH)
    vals = jnp.pad(vals, (0, pad)).reshape(NW, NCH, CH)

    x = jnp.pad(x0, ((0, NP - N), (0, 0)))
    total = x
    for _ in range(2):
        partial = _sc_layer(x, src, dst, vals)
        x, total = _combine(partial, total)
    return total[:USER], total[USER:N]


# speedup vs baseline: 3.4125x; 1.2520x over previous
import jax
import jax.numpy as jnp
from jax import lax
from jax.experimental import pallas as pl
from jax.experimental.pallas import tpu as pltpu
from jax.experimental.pallas import tpu_sc as plsc

USER = 6000
ITEM = 4000
N = USER + ITEM
E = 320000
D = 128
LEAKY = 0.5

NC = 2
NS = 16
NW = NC * NS
CH = 128
EPT = 10240
NCH = EPT // CH
EP = NW * EPT
NR = 2
NI = 4

NP = 10240
RPT = NP // NS
ZR = 32
NZ = RPT // ZR


def _sc_layer_body(x_hbm, src_hbm, dst_hbm, vals_hbm, part_hbm,
                   sc0, sc1, sc2, sc3, dc0, dc1, dc2, dc3,
                   vc0, vc1, vc2, vc3, r0, r1,
                   stage, accum,
                   sr0, sr1, sr2, sr3, sd0, sd1, sd2, sd3,
                   sv0, sv1, sv2, sv3, sg0, sg1, ss0, ss1):
    srcc = (sc0, sc1, sc2, sc3)
    dstc = (dc0, dc1, dc2, dc3)
    valsc = (vc0, vc1, vc2, vc3)
    rows = (r0, r1)
    sem_sr = (sr0, sr1, sr2, sr3)
    sem_d = (sd0, sd1, sd2, sd3)
    sem_v = (sv0, sv1, sv2, sv3)
    sem_g = (sg0, sg1)
    sem_s = (ss0, ss1)

    cid = lax.axis_index("c")
    sid = lax.axis_index("s")
    wid = sid * NC + cid
    rbase = sid * RPT

    def zrow(i, _):
        for k in range(8):
            stage[i, pl.ds(k * 16, 16)] = jnp.zeros((16,), jnp.float32)
        return 0
    lax.fori_loop(0, ZR, zrow, 0)
    for t in range(NZ):
        pltpu.sync_copy(stage, accum.at[pl.ds(rbase + t * ZR, ZR)])
    plsc.subcore_barrier()

    def fetch(j, bi):
        pltpu.async_copy(src_hbm.at[wid, j], srcc[bi], sem_sr[bi])
        pltpu.async_copy(dst_hbm.at[wid, j], dstc[bi], sem_d[bi])
        pltpu.async_copy(vals_hbm.at[wid, j], valsc[bi], sem_v[bi])

    def gather(j, bi, br):
        pltpu.make_async_copy(src_hbm.at[wid, j], srcc[bi], sem_sr[bi]).wait()
        pltpu.async_copy(x_hbm.at[srcc[bi]], rows[br], sem_g[br])

    fetch(0, 0)
    fetch(1, 1)
    gather(0, 0, 0)

    def scale(br, bi):
        def row16(g, _):
            base = g * 16
            vals16 = valsc[bi][pl.ds(base, 16)]
            for r in range(16):
                v = jnp.full((16,), vals16[r], jnp.float32)
                for k in range(8):
                    sl = pl.ds(k * 16, 16)
                    rows[br][base + r, sl] = rows[br][base + r, sl] * v
            return 0
        lax.fori_loop(0, CH // 16, row16, 0)

    def slot(j, b):
        br = b % NR
        bi = b % NI
        pltpu.make_async_copy(x_hbm.at[srcc[bi]], rows[br], sem_g[br]).wait()
        pltpu.make_async_copy(dst_hbm.at[wid, j], dstc[bi], sem_d[bi]).wait()
        pltpu.make_async_copy(vals_hbm.at[wid, j], valsc[bi], sem_v[bi]).wait()
        scale(br, bi)
        pltpu.async_copy(rows[br], accum.at[dstc[bi]], sem_s[br], add=True)
        @pl.when(j >= 1)
        def _():
            pltpu.make_async_copy(rows[1 - br], accum.at[dstc[(b + 3) % NI]],
                                  sem_s[1 - br]).wait()
        @pl.when(j + 2 < NCH)
        def _():
            fetch(j + 2, (b + 2) % NI)
        @pl.when(j + 1 < NCH)
        def _():
            gather(j + 1, (b + 1) % NI, 1 - br)

    def quad(i, _):
        for b in range(4):
            j = i * 4 + b
            slot(j, b)
        return 0
    lax.fori_loop(0, NCH // 4, quad, 0)
    pltpu.make_async_copy(rows[(NCH - 1) % NR],
                          accum.at[dstc[(NCH - 1) % NI]],
                          sem_s[(NCH - 1) % NR]).wait()
    plsc.subcore_barrier()

    for t in range(NZ):
        pltpu.sync_copy(accum.at[pl.ds(rbase + t * ZR, ZR)], stage)
        pltpu.sync_copy(stage, part_hbm.at[cid, pl.ds(rbase + t * ZR, ZR)])


_sc_layer = pl.kernel(
    _sc_layer_body,
    out_type=jax.ShapeDtypeStruct((NC, NP, D), jnp.float32),
    mesh=plsc.VectorSubcoreMesh(core_axis_name="c", subcore_axis_name="s",
                                num_cores=NC, num_subcores=NS),
    scratch_types=(
        [pltpu.VMEM((CH,), jnp.int32) for _ in range(NI)]
        + [pltpu.VMEM((CH,), jnp.int32) for _ in range(NI)]
        + [pltpu.VMEM((CH,), jnp.float32) for _ in range(NI)]
        + [pltpu.VMEM((CH, D), jnp.float32) for _ in range(NR)]
        + [
            pltpu.VMEM((ZR, D), jnp.float32),
            pltpu.VMEM_SHARED((NP, D), jnp.float32),
        ]
        + [pltpu.SemaphoreType.DMA for _ in range(3 * NI + 2 * NR)]
    ),
)


def _combine_body(p_ref, t_ref, y_ref, tot_ref):
    s = p_ref[0] + p_ref[1]
    y = jnp.maximum(s, LEAKY * s)
    y_ref[...] = y
    tot_ref[...] = t_ref[...] + y


_BR = 1024


def _combine(partial, total):
    return pl.pallas_call(
        _combine_body,
        grid=(NP // _BR,),
        in_specs=[
            pl.BlockSpec((NC, _BR, D), lambda i: (0, i, 0)),
            pl.BlockSpec((_BR, D), lambda i: (i, 0)),
        ],
        out_specs=[
            pl.BlockSpec((_BR, D), lambda i: (i, 0)),
            pl.BlockSpec((_BR, D), lambda i: (i, 0)),
        ],
        out_shape=[
            jax.ShapeDtypeStruct((NP, D), jnp.float32),
            jax.ShapeDtypeStruct((NP, D), jnp.float32),
        ],
    )(partial, total)


def kernel(uEmbeds, iEmbeds, edge_vals, edge_index):
    x0 = jnp.concatenate([uEmbeds, iEmbeds], axis=0)
    dst = edge_index[0].astype(jnp.int32)
    src = edge_index[1].astype(jnp.int32)
    vals = edge_vals.astype(jnp.float32)
    pad = EP - E
    src = jnp.pad(src, (0, pad)).reshape(NW, NCH, CH)
    dst = jnp.pad(dst, (0, pad)).reshape(NW, NCH, CH)
    vals = jnp.pad(vals, (0, pad)).reshape(NW, NCH, CH)

    x = jnp.pad(x0, ((0, NP - N), (0, 0)))
    total = x
    for _ in range(2):
        partial = _sc_layer(x, src, dst, vals)
        x, total = _combine(partial, total)
    return total[:USER], total[USER:N]


# speedup vs baseline: 3.4237x; 1.0033x over previous
import jax
import jax.numpy as jnp
from jax import lax
from jax.experimental import pallas as pl
from jax.experimental.pallas import tpu as pltpu
from jax.experimental.pallas import tpu_sc as plsc

USER = 6000
ITEM = 4000
N = USER + ITEM
E = 320000
D = 128
LEAKY = 0.5

NC = 2
NS = 16
NW = NC * NS
CH = 128
EPT = 10240
NCH = EPT // CH
EP = NW * EPT
NR = 2
NI = 4

NP = 10240
RPT = NP // NS


def _sc_layer_body(x_hbm, src_hbm, dst_hbm, vals_hbm, zeros_hbm, part_hbm,
                   sc0, sc1, sc2, sc3, dc0, dc1, dc2, dc3,
                   vc0, vc1, vc2, vc3, r0, r1,
                   accum,
                   sr0, sr1, sr2, sr3, sd0, sd1, sd2, sd3,
                   sv0, sv1, sv2, sv3, sg0, sg1, ss0, ss1, sz):
    srcc = (sc0, sc1, sc2, sc3)
    dstc = (dc0, dc1, dc2, dc3)
    valsc = (vc0, vc1, vc2, vc3)
    rows = (r0, r1)
    sem_sr = (sr0, sr1, sr2, sr3)
    sem_d = (sd0, sd1, sd2, sd3)
    sem_v = (sv0, sv1, sv2, sv3)
    sem_g = (sg0, sg1)
    sem_s = (ss0, ss1)

    cid = lax.axis_index("c")
    sid = lax.axis_index("s")
    wid = sid * NC + cid
    rbase = sid * RPT

    pltpu.async_copy(zeros_hbm.at[pl.ds(rbase, RPT)],
                     accum.at[pl.ds(rbase, RPT)], sz)

    def fetch(j, bi):
        pltpu.async_copy(src_hbm.at[wid, j], srcc[bi], sem_sr[bi])
        pltpu.async_copy(dst_hbm.at[wid, j], dstc[bi], sem_d[bi])
        pltpu.async_copy(vals_hbm.at[wid, j], valsc[bi], sem_v[bi])

    def gather(j, bi, br):
        pltpu.make_async_copy(src_hbm.at[wid, j], srcc[bi], sem_sr[bi]).wait()
        pltpu.async_copy(x_hbm.at[srcc[bi]], rows[br], sem_g[br])

    fetch(0, 0)
    fetch(1, 1)
    gather(0, 0, 0)
    pltpu.make_async_copy(zeros_hbm.at[pl.ds(rbase, RPT)],
                          accum.at[pl.ds(rbase, RPT)], sz).wait()
    plsc.subcore_barrier()

    def scale(br, bi):
        @plsc.parallel_loop(0, CH // 16, unroll=2)
        def _(g):
            base = g * 16
            vals16 = valsc[bi][pl.ds(base, 16)]
            for r in range(16):
                v = jnp.full((16,), vals16[r], jnp.float32)
                for k in range(8):
                    sl = pl.ds(k * 16, 16)
                    rows[br][base + r, sl] = rows[br][base + r, sl] * v

    def slot(j, b):
        br = b % NR
        bi = b % NI
        pltpu.make_async_copy(x_hbm.at[srcc[bi]], rows[br], sem_g[br]).wait()
        pltpu.make_async_copy(dst_hbm.at[wid, j], dstc[bi], sem_d[bi]).wait()
        pltpu.make_async_copy(vals_hbm.at[wid, j], valsc[bi], sem_v[bi]).wait()
        scale(br, bi)
        pltpu.async_copy(rows[br], accum.at[dstc[bi]], sem_s[br], add=True)
        @pl.when(j >= 1)
        def _():
            pltpu.make_async_copy(rows[1 - br], accum.at[dstc[(b + 3) % NI]],
                                  sem_s[1 - br]).wait()
        @pl.when(j + 2 < NCH)
        def _():
            fetch(j + 2, (b + 2) % NI)
        @pl.when(j + 1 < NCH)
        def _():
            gather(j + 1, (b + 1) % NI, 1 - br)

    def quad(i, _):
        for b in range(4):
            j = i * 4 + b
            slot(j, b)
        return 0
    lax.fori_loop(0, NCH // 4, quad, 0)
    pltpu.make_async_copy(rows[(NCH - 1) % NR],
                          accum.at[dstc[(NCH - 1) % NI]],
                          sem_s[(NCH - 1) % NR]).wait()
    plsc.subcore_barrier()

    pltpu.async_copy(accum.at[pl.ds(rbase, RPT)],
                     part_hbm.at[cid, pl.ds(rbase, RPT)], sz)
    pltpu.make_async_copy(accum.at[pl.ds(rbase, RPT)],
                          part_hbm.at[cid, pl.ds(rbase, RPT)], sz).wait()


_sc_layer = pl.kernel(
    _sc_layer_body,
    out_type=jax.ShapeDtypeStruct((NC, NP, D), jnp.float32),
    mesh=plsc.VectorSubcoreMesh(core_axis_name="c", subcore_axis_name="s",
                                num_cores=NC, num_subcores=NS),
    scratch_types=(
        [pltpu.VMEM((CH,), jnp.int32) for _ in range(NI)]
        + [pltpu.VMEM((CH,), jnp.int32) for _ in range(NI)]
        + [pltpu.VMEM((CH,), jnp.float32) for _ in range(NI)]
        + [pltpu.VMEM((CH, D), jnp.float32) for _ in range(NR)]
        + [
            pltpu.VMEM_SHARED((NP, D), jnp.float32),
        ]
        + [pltpu.SemaphoreType.DMA for _ in range(3 * NI + 2 * NR + 1)]
    ),
)


def _combine_body(p_ref, t_ref, y_ref, tot_ref):
    s = p_ref[0] + p_ref[1]
    y = jnp.maximum(s, LEAKY * s)
    y_ref[...] = y
    tot_ref[...] = t_ref[...] + y


_BR = 1024


def _combine(partial, total):
    return pl.pallas_call(
        _combine_body,
        grid=(NP // _BR,),
        in_specs=[
            pl.BlockSpec((NC, _BR, D), lambda i: (0, i, 0)),
            pl.BlockSpec((_BR, D), lambda i: (i, 0)),
        ],
        out_specs=[
            pl.BlockSpec((_BR, D), lambda i: (i, 0)),
            pl.BlockSpec((_BR, D), lambda i: (i, 0)),
        ],
        out_shape=[
            jax.ShapeDtypeStruct((NP, D), jnp.float32),
            jax.ShapeDtypeStruct((NP, D), jnp.float32),
        ],
    )(partial, total)


def kernel(uEmbeds, iEmbeds, edge_vals, edge_index):
    x0 = jnp.concatenate([uEmbeds, iEmbeds], axis=0)
    dst = edge_index[0].astype(jnp.int32)
    src = edge_index[1].astype(jnp.int32)
    vals = edge_vals.astype(jnp.float32)
    pad = EP - E
    src = jnp.pad(src, (0, pad)).reshape(NW, NCH, CH)
    dst = jnp.pad(dst, (0, pad)).reshape(NW, NCH, CH)
    vals = jnp.pad(vals, (0, pad)).reshape(NW, NCH, CH)

    x = jnp.pad(x0, ((0, NP - N), (0, 0)))
    zeros = jnp.zeros((NP, D), jnp.float32)
    total = x
    for _ in range(2):
        partial = _sc_layer(x, src, dst, vals, zeros)
        x, total = _combine(partial, total)
    return total[:USER], total[USER:N]


# speedup vs baseline: 3.4297x; 1.0018x over previous
import jax
import jax.numpy as jnp
from jax import lax
from jax.experimental import pallas as pl
from jax.experimental.pallas import tpu as pltpu
from jax.experimental.pallas import tpu_sc as plsc

USER = 6000
ITEM = 4000
N = USER + ITEM
E = 320000
D = 128
LEAKY = 0.5

NC = 2
NS = 16
NW = NC * NS
CH = 64
EPT = 10240
NCH = EPT // CH
EP = NW * EPT
NR = 4
NI = 8

NP = 10240
RPT = NP // NS

GLEAD = 3
FLEAD = 5


def _sc_layer_body(x_hbm, src_hbm, dst_hbm, vals_hbm, zeros_hbm, part_hbm,
                   sc0, sc1, sc2, sc3, sc4, sc5, sc6, sc7,
                   dc0, dc1, dc2, dc3, dc4, dc5, dc6, dc7,
                   vc0, vc1, vc2, vc3, vc4, vc5, vc6, vc7,
                   r0, r1, r2, r3,
                   accum,
                   sr0, sr1, sr2, sr3, sr4, sr5, sr6, sr7,
                   sd0, sd1, sd2, sd3, sd4, sd5, sd6, sd7,
                   sv0, sv1, sv2, sv3, sv4, sv5, sv6, sv7,
                   sg0, sg1, sg2, sg3, ss0, ss1, sz):
    srcc = (sc0, sc1, sc2, sc3, sc4, sc5, sc6, sc7)
    dstc = (dc0, dc1, dc2, dc3, dc4, dc5, dc6, dc7)
    valsc = (vc0, vc1, vc2, vc3, vc4, vc5, vc6, vc7)
    rows = (r0, r1, r2, r3)
    sem_sr = (sr0, sr1, sr2, sr3, sr4, sr5, sr6, sr7)
    sem_d = (sd0, sd1, sd2, sd3, sd4, sd5, sd6, sd7)
    sem_v = (sv0, sv1, sv2, sv3, sv4, sv5, sv6, sv7)
    sem_g = (sg0, sg1, sg2, sg3)
    sem_s = (ss0, ss1)

    cid = lax.axis_index("c")
    sid = lax.axis_index("s")
    wid = sid * NC + cid
    rbase = sid * RPT

    def fetch(j, bi):
        pltpu.async_copy(src_hbm.at[wid, j], srcc[bi], sem_sr[bi])
        pltpu.async_copy(dst_hbm.at[wid, j], dstc[bi], sem_d[bi])
        pltpu.async_copy(vals_hbm.at[wid, j], valsc[bi], sem_v[bi])

    def gather(j, bi, br):
        pltpu.make_async_copy(src_hbm.at[wid, j], srcc[bi], sem_sr[bi]).wait()
        pltpu.async_copy(x_hbm.at[srcc[bi]], rows[br], sem_g[br])

    pltpu.async_copy(zeros_hbm.at[pl.ds(rbase, RPT)],
                     accum.at[pl.ds(rbase, RPT)], sz)
    for j in range(FLEAD):
        fetch(j, j)
    for j in range(GLEAD):
        gather(j, j, j)
    pltpu.make_async_copy(zeros_hbm.at[pl.ds(rbase, RPT)],
                          accum.at[pl.ds(rbase, RPT)], sz).wait()
    plsc.subcore_barrier()

    def scale(br, bi):
        @plsc.parallel_loop(0, CH // 16, unroll=2)
        def _(g):
            base = g * 16
            vals16 = valsc[bi][pl.ds(base, 16)]
            for r in range(16):
                v = jnp.full((16,), vals16[r], jnp.float32)
                for k in range(D // 16):
                    sl = pl.ds(k * 16, 16)
                    rows[br][base + r, sl] = rows[br][base + r, sl] * v

    def slot(j, b):
        br = b % NR
        bs = b % 2
        @pl.when(j >= 1)
        def _():
            pltpu.make_async_copy(rows[(b + NR - 1) % NR],
                                  accum.at[dstc[(b + NI - 1) % NI]],
                                  sem_s[1 - bs]).wait()
        @pl.when(j + GLEAD < NCH)
        def _():
            gather(j + GLEAD, (b + GLEAD) % NI, (b + GLEAD) % NR)
        pltpu.make_async_copy(x_hbm.at[srcc[b]], rows[br], sem_g[br]).wait()
        pltpu.make_async_copy(vals_hbm.at[wid, j], valsc[b], sem_v[b]).wait()
        scale(br, b)
        pltpu.make_async_copy(dst_hbm.at[wid, j], dstc[b], sem_d[b]).wait()
        pltpu.async_copy(rows[br], accum.at[dstc[b]], sem_s[bs], add=True)
        @pl.when(j + FLEAD < NCH)
        def _():
            fetch(j + FLEAD, (b + FLEAD) % NI)

    def octet(i, _):
        for b in range(NI):
            j = i * NI + b
            slot(j, b)
        return 0
    lax.fori_loop(0, NCH // NI, octet, 0)
    pltpu.make_async_copy(rows[(NCH - 1) % NR],
                          accum.at[dstc[(NCH - 1) % NI]],
                          sem_s[(NCH - 1) % 2]).wait()
    plsc.subcore_barrier()

    pltpu.async_copy(accum.at[pl.ds(rbase, RPT)],
                     part_hbm.at[cid, pl.ds(rbase, RPT)], sz)
    pltpu.make_async_copy(accum.at[pl.ds(rbase, RPT)],
                          part_hbm.at[cid, pl.ds(rbase, RPT)], sz).wait()


_sc_layer = pl.kernel(
    _sc_layer_body,
    out_type=jax.ShapeDtypeStruct((NC, NP, D), jnp.float32),
    mesh=plsc.VectorSubcoreMesh(core_axis_name="c", subcore_axis_name="s",
                                num_cores=NC, num_subcores=NS),
    scratch_types=(
        [pltpu.VMEM((CH,), jnp.int32) for _ in range(NI)]
        + [pltpu.VMEM((CH,), jnp.int32) for _ in range(NI)]
        + [pltpu.VMEM((CH,), jnp.float32) for _ in range(NI)]
        + [pltpu.VMEM((CH, D), jnp.float32) for _ in range(NR)]
        + [
            pltpu.VMEM_SHARED((NP, D), jnp.float32),
        ]
        + [pltpu.SemaphoreType.DMA for _ in range(3 * NI + NR + 2 + 1)]
    ),
)


def _combine_body(p_ref, t_ref, y_ref, tot_ref):
    s = p_ref[0] + p_ref[1]
    y = jnp.maximum(s, LEAKY * s)
    y_ref[...] = y
    tot_ref[...] = t_ref[...] + y


_BR = 1024


def _combine(partial, total):
    return pl.pallas_call(
        _combine_body,
        grid=(NP // _BR,),
        in_specs=[
            pl.BlockSpec((NC, _BR, D), lambda i: (0, i, 0)),
            pl.BlockSpec((_BR, D), lambda i: (i, 0)),
        ],
        out_specs=[
            pl.BlockSpec((_BR, D), lambda i: (i, 0)),
            pl.BlockSpec((_BR, D), lambda i: (i, 0)),
        ],
        out_shape=[
            jax.ShapeDtypeStruct((NP, D), jnp.float32),
            jax.ShapeDtypeStruct((NP, D), jnp.float32),
        ],
    )(partial, total)


def kernel(uEmbeds, iEmbeds, edge_vals, edge_index):
    x0 = jnp.concatenate([uEmbeds, iEmbeds], axis=0)
    dst = edge_index[0].astype(jnp.int32)
    src = edge_index[1].astype(jnp.int32)
    vals = edge_vals.astype(jnp.float32)
    pad = EP - E
    src = jnp.pad(src, (0, pad)).reshape(NW, NCH, CH)
    dst = jnp.pad(dst, (0, pad)).reshape(NW, NCH, CH)
    vals = jnp.pad(vals, (0, pad)).reshape(NW, NCH, CH)

    x = jnp.pad(x0, ((0, NP - N), (0, 0)))
    zeros = jnp.zeros((NP, D), jnp.float32)
    total = x
    for _ in range(2):
        partial = _sc_layer(x, src, dst, vals, zeros)
        x, total = _combine(partial, total)
    return total[:USER], total[USER:N]
